# baseline (speedup 1.0000x reference)
import jax
import jax.numpy as jnp
from jax import lax
from jax.experimental import pallas as pl

_ANCH_W = (10.0, 16.0, 33.0)
_ANCH_H = (13.0, 30.0, 23.0)
_GS = 52
_G = _GS * _GS
_NA = 3
_NF = 85
_STRIDE = 8.0


def _body(x_ref, o_ref):
    a = pl.program_id(1)
    v = x_ref[0, 0]

    aw = jnp.where(a == 0, _ANCH_W[0], jnp.where(a == 1, _ANCH_W[1], _ANCH_W[2]))
    ah = jnp.where(a == 0, _ANCH_H[0], jnp.where(a == 1, _ANCH_H[1], _ANCH_H[2]))

    shp = (_GS, _GS, _NF)
    k = lax.broadcasted_iota(jnp.int32, shp, 2)
    gx = lax.broadcasted_iota(jnp.int32, shp, 1).astype(jnp.float32)
    gy = lax.broadcasted_iota(jnp.int32, shp, 0).astype(jnp.float32)

    sig = jax.nn.sigmoid(v)
    ex = jnp.exp(v)
    is_wh = (k == 2) | (k == 3)
    base = jnp.where(is_wh, ex, sig)
    scale = jnp.where(k < 2, _STRIDE,
                      jnp.where(k == 2, aw, jnp.where(k == 3, ah, 1.0)))
    grid_term = jnp.where(k == 0, gx, jnp.where(k == 1, gy, 0.0))
    o_ref[0, 0] = base * scale + grid_term * _STRIDE


def kernel(inputs):
    b = inputs.shape[0]
    x = jnp.transpose(inputs.reshape(b, _NA, _NF, _GS, _GS), (0, 1, 3, 4, 2))
    out = pl.pallas_call(
        _body,
        grid=(b, _NA),
        in_specs=[pl.BlockSpec((1, 1, _GS, _GS, _NF), lambda i, j: (i, j, 0, 0, 0))],
        out_specs=pl.BlockSpec((1, 1, _GS, _GS, _NF), lambda i, j: (i, j, 0, 0, 0)),
        out_shape=jax.ShapeDtypeStruct((b, _NA, _GS, _GS, _NF), jnp.float32),
    )(x)
    return (out.reshape(b, _NA * _G, _NF), 0, 0)
